# baseline (speedup 1.0000x reference)
import functools

import jax
import jax.numpy as jnp
from jax import lax
from jax.experimental import pallas as pl
from jax.experimental.pallas import tpu as pltpu
from jax.experimental.pallas import tpu_sc as plsc

B = 16384
L = 50
EMBD = 64
NTOT = B * L
BLK = 128
NBLK = NTOT // BLK
NW = 32
BPW = NBLK // NW
NBUF = 4
NBC = B // BLK

_mesh = plsc.VectorSubcoreMesh(core_axis_name="c", subcore_axis_name="s")


@functools.partial(
    pl.kernel,
    mesh=_mesh,
    out_type=jax.ShapeDtypeStruct((L, EMBD // 8, NBC, 8, BLK), jnp.float32),
    scratch_types=[
        pltpu.VMEM((BPW, BLK), jnp.int32),
        [pltpu.VMEM((BLK, EMBD), jnp.float32) for _ in range(NBUF)],
        [pltpu.VMEM((EMBD, BLK), jnp.float32) for _ in range(NBUF)],
        pltpu.SemaphoreType.DMA((NBUF,)),
        pltpu.SemaphoreType.DMA((NBUF,)),
    ],
    compiler_params=pltpu.CompilerParams(
        use_tc_tiling_on_sc=False, needs_layout_passes=False),
)
def _gather_kernel(idx_hbm, table_hbm, out_hbm, idx_v, rows_v,
                   trans_v, gsem, ssem):
    wid = lax.axis_index("s") * 2 + lax.axis_index("c")
    base = wid * BPW

    pltpu.sync_copy(idx_hbm.at[pl.ds(base, BPW)], idx_v)

    def start_gather(b, t):
        pltpu.async_copy(table_hbm.at[idx_v.at[t]], rows_v[b], gsem.at[b])

    def wait_gather(b, t):
        pltpu.make_async_copy(
            table_hbm.at[idx_v.at[t]], rows_v[b], gsem.at[b]).wait()

    def block_lbc(t):
        blk = base + t
        return blk // NBC, blk % NBC

    def start_stores(b, t):
        l, bc = block_lbc(t)
        for er in range(EMBD // 8):
            pltpu.async_copy(
                trans_v[b].at[pl.ds(er * 8, 8)], out_hbm.at[l, er, bc],
                ssem.at[b])

    def wait_stores(b, t):
        l, bc = block_lbc(t)
        for er in range(EMBD // 8):
            pltpu.make_async_copy(
                trans_v[b].at[pl.ds(er * 8, 8)], out_hbm.at[l, er, bc],
                ssem.at[b]).wait()

    def transpose(b, t):
        def cgroup(g, carry):
            c0 = g * 16
            row_idx = c0 + lax.iota(jnp.int32, 16)
            for e in range(EMBD):
                col_idx = jnp.full((16,), e, jnp.int32)
                vals = plsc.load_gather(rows_v[b], [row_idx, col_idx])
                trans_v[b][e, pl.ds(c0, 16)] = vals
            return carry
        lax.fori_loop(0, BLK // 16, cgroup, 0)

    for b in range(NBUF):
        start_gather(b, b)
    for k in range(NBUF):
        wait_gather(k, k)
        transpose(k, k)
        start_stores(k, k)
        start_gather(k, k + NBUF)

    def group(j, carry):
        for k in range(NBUF):
            t = NBUF * j + k
            b = k
            wait_gather(b, t)
            wait_stores(b, t - NBUF)
            transpose(b, t)
            start_stores(b, t)
            start_gather(b, t + NBUF)
        return carry

    lax.fori_loop(1, BPW // NBUF - 1, group, 0)

    for k in range(NBUF):
        t = BPW - NBUF + k
        wait_gather(k, t)
        wait_stores(k, t - NBUF)
        transpose(k, t)
        start_stores(k, t)
    for k in range(NBUF):
        wait_stores(k, BPW - NBUF + k)


def kernel(x, table):
    idx = x.T.reshape(NBLK, BLK)
    out5 = _gather_kernel(idx, table)
    return out5.transpose(2, 4, 0, 1, 3).reshape(B, L, EMBD)


# speedup vs baseline: 1.3800x; 1.3800x over previous
import functools

import jax
import jax.numpy as jnp
from jax import lax
from jax.experimental import pallas as pl
from jax.experimental.pallas import tpu as pltpu
from jax.experimental.pallas import tpu_sc as plsc

B = 16384
L = 50
EMBD = 64
NTOT = B * L
BLK = 128
NBLK = NTOT // BLK
NW = 32
BPW = NBLK // NW
NBUF = 4
NBC = B // BLK

_mesh = plsc.VectorSubcoreMesh(core_axis_name="c", subcore_axis_name="s")


@functools.partial(
    pl.kernel,
    mesh=_mesh,
    out_type=jax.ShapeDtypeStruct((L, EMBD // 8, NBC, 8, BLK), jnp.float32),
    scratch_types=[
        pltpu.VMEM((BPW, BLK), jnp.int32),
        [pltpu.VMEM((BLK, EMBD), jnp.float32) for _ in range(NBUF)],
        [pltpu.VMEM((EMBD, BLK), jnp.float32) for _ in range(NBUF)],
        pltpu.SemaphoreType.DMA((NBUF,)),
        pltpu.SemaphoreType.DMA((NBUF,)),
    ],
    compiler_params=pltpu.CompilerParams(
        use_tc_tiling_on_sc=False, needs_layout_passes=False),
)
def _gather_kernel(idx_hbm, table_hbm, out_hbm, idx_v, rows_v,
                   trans_v, gsem, ssem):
    wid = lax.axis_index("s") * 2 + lax.axis_index("c")
    base = wid * BPW

    pltpu.sync_copy(idx_hbm.at[pl.ds(base, BPW)], idx_v)

    def start_gather(b, t):
        pltpu.async_copy(table_hbm.at[idx_v.at[t]], rows_v[b], gsem.at[b])

    def wait_gather(b, t):
        pltpu.make_async_copy(
            table_hbm.at[idx_v.at[t]], rows_v[b], gsem.at[b]).wait()

    def block_lbc(t):
        blk = base + t
        return blk // NBC, blk % NBC

    def start_stores(b, t):
        l, bc = block_lbc(t)
        for er in range(EMBD // 8):
            pltpu.async_copy(
                trans_v[b].at[pl.ds(er * 8, 8)], out_hbm.at[l, er, bc],
                ssem.at[b])

    def wait_stores(b, t):
        l, bc = block_lbc(t)
        for er in range(EMBD // 8):
            pltpu.make_async_copy(
                trans_v[b].at[pl.ds(er * 8, 8)], out_hbm.at[l, er, bc],
                ssem.at[b]).wait()

    def transpose(b, t):
        def cgroup(g, carry):
            c0 = g * 16
            row_idx = c0 + lax.iota(jnp.int32, 16)
            for e0 in range(0, EMBD, 8):
                vals = []
                for i in range(8):
                    col_idx = jnp.full((16,), e0 + i, jnp.int32)
                    vals.append(
                        plsc.load_gather(rows_v[b], [row_idx, col_idx]))
                for i in range(8):
                    trans_v[b][e0 + i, pl.ds(c0, 16)] = vals[i]
            return carry
        lax.fori_loop(0, BLK // 16, cgroup, 0)

    for b in range(NBUF):
        start_gather(b, b)
    for k in range(NBUF):
        wait_gather(k, k)
        transpose(k, k)
        start_stores(k, k)
        start_gather(k, k + NBUF)

    def group(j, carry):
        for k in range(NBUF):
            t = NBUF * j + k
            b = k
            wait_gather(b, t)
            wait_stores(b, t - NBUF)
            transpose(b, t)
            start_stores(b, t)
            start_gather(b, t + NBUF)
        return carry

    lax.fori_loop(1, BPW // NBUF - 1, group, 0)

    for k in range(NBUF):
        t = BPW - NBUF + k
        wait_gather(k, t)
        wait_stores(k, t - NBUF)
        transpose(k, t)
        start_stores(k, t)
    for k in range(NBUF):
        wait_stores(k, BPW - NBUF + k)


def kernel(x, table):
    idx = x.T.reshape(NBLK, BLK)
    out5 = _gather_kernel(idx, table)
    return out5.transpose(2, 4, 0, 1, 3).reshape(B, L, EMBD)


# speedup vs baseline: 2.1058x; 1.5259x over previous
import functools

import jax
import jax.numpy as jnp
from jax import lax
from jax.experimental import pallas as pl
from jax.experimental.pallas import tpu as pltpu
from jax.experimental.pallas import tpu_sc as plsc

B = 16384
L = 50
EMBD = 64
NTOT = B * L
BLK = 128
NBLK = NTOT // BLK
NW = 32
BPW = NBLK // NW
NBUF = 4
NBC = B // BLK

_mesh = plsc.VectorSubcoreMesh(core_axis_name="c", subcore_axis_name="s")


@functools.partial(
    pl.kernel,
    mesh=_mesh,
    out_type=jax.ShapeDtypeStruct((L, EMBD // 8, NBC, 8, BLK), jnp.float32),
    scratch_types=[
        pltpu.VMEM((BPW, BLK), jnp.int32),
        [pltpu.VMEM((BLK, EMBD), jnp.float32) for _ in range(NBUF)],
        [pltpu.VMEM((EMBD, BLK + 1), jnp.float32) for _ in range(NBUF)],
        pltpu.SemaphoreType.DMA((NBUF,)),
        pltpu.SemaphoreType.DMA((NBUF,)),
    ],
    compiler_params=pltpu.CompilerParams(
        use_tc_tiling_on_sc=False, needs_layout_passes=False),
)
def _gather_kernel(idx_hbm, table_hbm, out_hbm, idx_v, rows_v,
                   trans_v, gsem, ssem):
    wid = lax.axis_index("s") * 2 + lax.axis_index("c")
    base = wid * BPW

    pltpu.sync_copy(idx_hbm.at[pl.ds(base, BPW)], idx_v)

    def start_gather(b, t):
        pltpu.async_copy(table_hbm.at[idx_v.at[t]], rows_v[b], gsem.at[b])

    def wait_gather(b, t):
        pltpu.make_async_copy(
            table_hbm.at[idx_v.at[t]], rows_v[b], gsem.at[b]).wait()

    def block_lbc(t):
        blk = base + t
        return blk // NBC, blk % NBC

    def start_stores(b, t):
        l, bc = block_lbc(t)
        for er in range(EMBD // 8):
            pltpu.async_copy(
                trans_v[b].at[pl.ds(er * 8, 8), pl.ds(0, BLK)],
                out_hbm.at[l, er, bc], ssem.at[b])

    def wait_stores(b, t):
        l, bc = block_lbc(t)
        for er in range(EMBD // 8):
            pltpu.make_async_copy(
                trans_v[b].at[pl.ds(er * 8, 8), pl.ds(0, BLK)],
                out_hbm.at[l, er, bc], ssem.at[b]).wait()

    def transpose(b, t):
        e_idx = [j * 16 + lax.iota(jnp.int32, 16) for j in range(EMBD // 16)]

        def cols(c0, carry):
            vals = []
            cidx = []
            for u in range(4):
                c = c0 * 4 + u
                cidx.append(jnp.full((16,), c, jnp.int32))
                for j in range(EMBD // 16):
                    vals.append(rows_v[b][c, pl.ds(j * 16, 16)])
            for u in range(4):
                for j in range(EMBD // 16):
                    plsc.store_scatter(
                        trans_v[b], [e_idx[j], cidx[u]],
                        vals[u * (EMBD // 16) + j])
            return carry
        lax.fori_loop(0, BLK // 4, cols, 0)

    for b in range(NBUF):
        start_gather(b, b)
    for k in range(NBUF):
        wait_gather(k, k)
        transpose(k, k)
        start_stores(k, k)
        start_gather(k, k + NBUF)

    def group(j, carry):
        for k in range(NBUF):
            t = NBUF * j + k
            b = k
            wait_gather(b, t)
            wait_stores(b, t - NBUF)
            transpose(b, t)
            start_stores(b, t)
            start_gather(b, t + NBUF)
        return carry

    lax.fori_loop(1, BPW // NBUF - 1, group, 0)

    for k in range(NBUF):
        t = BPW - NBUF + k
        wait_gather(k, t)
        wait_stores(k, t - NBUF)
        transpose(k, t)
        start_stores(k, t)
    for k in range(NBUF):
        wait_stores(k, BPW - NBUF + k)


def kernel(x, table):
    idx = x.T.reshape(NBLK, BLK)
    out5 = _gather_kernel(idx, table)
    return out5.transpose(2, 4, 0, 1, 3).reshape(B, L, EMBD)
